# initial kernel scaffold (unmeasured)
import jax
import jax.numpy as jnp
from jax import lax
from jax.experimental import pallas as pl
from jax.experimental.pallas import tpu as pltpu

N_DEV = 4


def _allgather_body(
    x_ref, d_ref,
    xout_ref, dout_ref,
    xcomm, dcomm,
    xsend_sems, xrecv_sems, dsend_sems, drecv_sems,
):
    my_pos = lax.axis_index("i")
    right = lax.rem(my_pos + 1, N_DEV)

    m_per = x_ref.shape[0]
    dm_per = d_ref.shape[0]

    xout_ref[pl.ds(my_pos * m_per, m_per), :] = x_ref[:, :]
    dout_ref[pl.ds(my_pos * dm_per, dm_per), :] = d_ref[:, :]
    xcomm[0, :, :] = x_ref[:, :]
    dcomm[0, :, :] = d_ref[:, :]

    for h in range(N_DEV - 1):
        send_slot = h % 2
        recv_slot = (h + 1) % 2
        xrdma = pltpu.make_async_remote_copy(
            src_ref=xcomm.at[send_slot],
            dst_ref=xcomm.at[recv_slot],
            send_sem=xsend_sems.at[send_slot],
            recv_sem=xrecv_sems.at[recv_slot],
            device_id=(right,),
            device_id_type=pl.DeviceIdType.MESH,
        )
        drdma = pltpu.make_async_remote_copy(
            src_ref=dcomm.at[send_slot],
            dst_ref=dcomm.at[recv_slot],
            send_sem=dsend_sems.at[send_slot],
            recv_sem=drecv_sems.at[recv_slot],
            device_id=(right,),
            device_id_type=pl.DeviceIdType.MESH,
        )
        xrdma.start()
        drdma.start()
        xrdma.wait()
        drdma.wait()

        origin = lax.rem(my_pos - h - 1 + N_DEV, N_DEV)
        xout_ref[pl.ds(origin * m_per, m_per), :] = xcomm[recv_slot, :, :]
        dout_ref[pl.ds(origin * dm_per, dm_per), :] = dcomm[recv_slot, :, :]


def kernel(x, dest):
    m_per, n = x.shape
    x16 = x.astype(jnp.bfloat16)
    d2d = dest.astype(jnp.int32).reshape(-1, 128)
    dm_per = d2d.shape[0]

    x_all, d_all = pl.pallas_call(
        _allgather_body,
        out_shape=(
            jax.ShapeDtypeStruct((N_DEV * m_per, n), jnp.bfloat16),
            jax.ShapeDtypeStruct((N_DEV * dm_per, 128), jnp.int32),
        ),
        in_specs=[
            pl.BlockSpec(memory_space=pltpu.VMEM),
            pl.BlockSpec(memory_space=pltpu.VMEM),
        ],
        out_specs=(
            pl.BlockSpec(memory_space=pltpu.VMEM),
            pl.BlockSpec(memory_space=pltpu.VMEM),
        ),
        scratch_shapes=[
            pltpu.VMEM((2, m_per, n), jnp.bfloat16),
            pltpu.VMEM((2, dm_per, 128), jnp.int32),
            pltpu.SemaphoreType.DMA((2,)),
            pltpu.SemaphoreType.DMA((2,)),
            pltpu.SemaphoreType.DMA((2,)),
            pltpu.SemaphoreType.DMA((2,)),
        ],
        compiler_params=pltpu.CompilerParams(collective_id=0),
    )(x16, d2d)

    me = lax.axis_index("i")
    dest_all = d_all.reshape(-1)
    order = jnp.argsort(dest_all, stable=True)
    mine = lax.dynamic_slice(order, (me * m_per,), (m_per,))
    return x_all[mine].astype(jnp.float32)


# baseline (device time: 169608 ns/iter reference)
import jax
import jax.numpy as jnp
from jax import lax
from jax.experimental import pallas as pl
from jax.experimental.pallas import tpu as pltpu

N_DEV = 4


def _allgather_body(
    x_ref, d_ref,
    xout_ref, dout_ref,
    xcomm, dcomm,
    xsend_sems, xrecv_sems, dsend_sems, drecv_sems,
):
    my_pos = lax.axis_index("i")
    right = lax.rem(my_pos + 1, N_DEV)

    m_per = x_ref.shape[0]
    dm_per = d_ref.shape[0]

    xout_ref[pl.ds(my_pos * m_per, m_per), :] = x_ref[:, :]
    dout_ref[pl.ds(my_pos * dm_per, dm_per), :] = d_ref[:, :]
    xcomm[0, :, :] = x_ref[:, :]
    dcomm[0, :, :] = d_ref[:, :]

    for h in range(N_DEV - 1):
        send_slot = h % 2
        recv_slot = (h + 1) % 2
        xrdma = pltpu.make_async_remote_copy(
            src_ref=xcomm.at[send_slot],
            dst_ref=xcomm.at[recv_slot],
            send_sem=xsend_sems.at[send_slot],
            recv_sem=xrecv_sems.at[recv_slot],
            device_id=(right,),
            device_id_type=pl.DeviceIdType.MESH,
        )
        drdma = pltpu.make_async_remote_copy(
            src_ref=dcomm.at[send_slot],
            dst_ref=dcomm.at[recv_slot],
            send_sem=dsend_sems.at[send_slot],
            recv_sem=drecv_sems.at[recv_slot],
            device_id=(right,),
            device_id_type=pl.DeviceIdType.MESH,
        )
        xrdma.start()
        drdma.start()
        xrdma.wait()
        drdma.wait()

        origin = lax.rem(my_pos - h - 1 + N_DEV, N_DEV)
        xout_ref[pl.ds(origin * m_per, m_per), :] = xcomm[recv_slot, :, :]
        dout_ref[pl.ds(origin * dm_per, dm_per), :] = dcomm[recv_slot, :, :]


def kernel(x, dest):
    m_per, n = x.shape
    x16 = x.astype(jnp.bfloat16)
    d2d = dest.astype(jnp.int32).reshape(-1, 128)
    dm_per = d2d.shape[0]

    x_all, d_all = pl.pallas_call(
        _allgather_body,
        out_shape=(
            jax.ShapeDtypeStruct((N_DEV * m_per, n), jnp.bfloat16),
            jax.ShapeDtypeStruct((N_DEV * dm_per, 128), jnp.int32),
        ),
        in_specs=[
            pl.BlockSpec(memory_space=pltpu.VMEM),
            pl.BlockSpec(memory_space=pltpu.VMEM),
        ],
        out_specs=(
            pl.BlockSpec(memory_space=pltpu.VMEM),
            pl.BlockSpec(memory_space=pltpu.VMEM),
        ),
        scratch_shapes=[
            pltpu.VMEM((2, m_per, n), jnp.bfloat16),
            pltpu.VMEM((2, dm_per, 128), jnp.int32),
            pltpu.SemaphoreType.DMA((2,)),
            pltpu.SemaphoreType.DMA((2,)),
            pltpu.SemaphoreType.DMA((2,)),
            pltpu.SemaphoreType.DMA((2,)),
        ],
    )(x16, d2d)

    me = lax.axis_index("i")
    dest_all = d_all.reshape(-1)
    order = jnp.argsort(dest_all, stable=True)
    mine = lax.dynamic_slice(order, (me * m_per,), (m_per,))
    return x_all[mine].astype(jnp.float32)


# device time: 102982 ns/iter; 1.6470x vs baseline; 1.6470x over previous
import jax
import jax.numpy as jnp
from jax import lax
from jax.experimental import pallas as pl
from jax.experimental.pallas import tpu as pltpu

N_DEV = 4
B = 64
MAXF = 10
MAXC = MAXF * B
MAXRECV = (N_DEV - 1) * MAXF


def _cnt_exchange_body(cnt_ref, all_ref, send_sem, recv_sem):
    my = lax.axis_index("i")
    all_ref[0, :, :] = cnt_ref[:, :]
    rdmas = []
    for d in range(1, N_DEV):
        peer = lax.rem(my + d, N_DEV)
        rdma = pltpu.make_async_remote_copy(
            src_ref=cnt_ref,
            dst_ref=all_ref.at[d],
            send_sem=send_sem,
            recv_sem=recv_sem,
            device_id=(peer,),
            device_id_type=pl.DeviceIdType.MESH,
        )
        rdma.start()
        rdmas.append(rdma)
    for rdma in rdmas:
        rdma.wait_send()
    for rdma in rdmas:
        rdma.wait_recv()


def _a2av_body(xs_ref, scal_ref, stage_ref, send_sem, recv_sem, loc_sem):
    my = lax.axis_index("i")

    send_preds = []
    for d in range(1, N_DEV):
        r = lax.rem(my + d, N_DEV)
        nb = scal_ref[r]
        for k in range(MAXF):
            pred = k < nb

            def _issue(k=k, r=r):
                rdma = pltpu.make_async_remote_copy(
                    src_ref=xs_ref.at[r * MAXF + k],
                    dst_ref=stage_ref.at[my * MAXF + k],
                    send_sem=send_sem,
                    recv_sem=recv_sem,
                    device_id=(r,),
                    device_id_type=pl.DeviceIdType.MESH,
                )
                rdma.start()

            pl.when(pred)(_issue)
            send_preds.append(pred)

    loc_copies = []
    for k in range(MAXF):
        cp = pltpu.make_async_copy(
            xs_ref.at[my * MAXF + k],
            stage_ref.at[my * MAXF + k],
            loc_sem,
        )
        cp.start()
        loc_copies.append(cp)

    dummy = pltpu.make_async_remote_copy(
        src_ref=xs_ref.at[0],
        dst_ref=stage_ref.at[0],
        send_sem=send_sem,
        recv_sem=recv_sem,
        device_id=(my,),
        device_id_type=pl.DeviceIdType.MESH,
    )
    for pred in send_preds:
        pl.when(pred)(dummy.wait_send)
    for cp in loc_copies:
        cp.wait()

    nrecv = scal_ref[N_DEV]
    for k in range(MAXRECV):
        pl.when(k < nrecv)(dummy.wait_recv)


def kernel(x, dest):
    m_per, n = x.shape
    dest = dest.astype(jnp.int32)

    perm = jnp.argsort(dest, stable=True)
    cnt_me = (dest[:, None] == jnp.arange(N_DEV)[None, :]).sum(
        axis=0, dtype=jnp.int32
    )
    loc_start = jnp.cumsum(cnt_me) - cnt_me

    q = jnp.arange(N_DEV * MAXC, dtype=jnp.int32)
    r_of = q // MAXC
    j_of = q % MAXC
    t = jnp.clip(loc_start[r_of] + j_of, 0, m_per - 1)
    x_send = (
        jnp.take(x, perm[t], axis=0)
        .astype(jnp.bfloat16)
        .reshape(N_DEV * MAXF, B, n)
    )

    cnt_row = jnp.zeros((8, 128), jnp.int32).at[0, :N_DEV].set(cnt_me)
    cnt_all = pl.pallas_call(
        _cnt_exchange_body,
        out_shape=jax.ShapeDtypeStruct((N_DEV, 8, 128), jnp.int32),
        in_specs=[pl.BlockSpec(memory_space=pltpu.VMEM)],
        out_specs=pl.BlockSpec(memory_space=pltpu.VMEM),
        scratch_shapes=[
            pltpu.SemaphoreType.DMA,
            pltpu.SemaphoreType.DMA,
        ],
    )(cnt_row)

    me = lax.axis_index("i")
    C = jnp.take(
        cnt_all[:, 0, :N_DEV], (me - jnp.arange(N_DEV)) % N_DEV, axis=0
    )
    nb_send = (cnt_me + B - 1) // B
    blocks_in = (jnp.take(C, me, axis=1) + B - 1) // B
    nrecv = jnp.sum(blocks_in) - jnp.take(blocks_in, me)
    scal = jnp.concatenate([nb_send, nrecv[None]]).astype(jnp.int32)

    stage = pl.pallas_call(
        _a2av_body,
        out_shape=jax.ShapeDtypeStruct((N_DEV * MAXF, B, n), jnp.bfloat16),
        in_specs=[
            pl.BlockSpec(memory_space=pltpu.VMEM),
            pl.BlockSpec(memory_space=pltpu.SMEM),
        ],
        out_specs=pl.BlockSpec(memory_space=pltpu.VMEM),
        scratch_shapes=[
            pltpu.SemaphoreType.DMA,
            pltpu.SemaphoreType.DMA,
            pltpu.SemaphoreType.DMA,
        ],
    )(x_send, scal)

    col = jnp.take(C, me, axis=1)
    off_excl = jnp.cumsum(col) - col
    j = jnp.arange(m_per, dtype=jnp.int32)
    s_j = jnp.searchsorted(off_excl, j, side="right").astype(jnp.int32) - 1
    within = j - off_excl[s_j]
    out16 = jnp.take(
        stage.reshape(N_DEV * MAXC, n), s_j * MAXC + within, axis=0
    )
    return out16.astype(jnp.float32)
